# Initial kernel scaffold; baseline (speedup 1.0000x reference)
import jax
import jax.numpy as jnp
from jax.experimental import pallas as pl


def kernel(x, embed):
    raise NotImplementedError("write your pallas kernel here")


# speedup vs baseline: 1.0560x; 1.0560x over previous
import jax
import jax.numpy as jnp
from jax import lax
from jax.experimental import pallas as pl
from jax.experimental.pallas import tpu as pltpu
from jax.experimental.pallas import tpu_sc as plsc

N = 32768
D = 32
K = 8192
NB = 512
KC = 1024
GRID = N // NB

_SC_NW = 32
_BPW = N // _SC_NW
_ICH = 128
_NCH = _BPW // _ICH


def _argmin_body(x_ref, et_ref, ind_ref, loss_ref):
    x = x_ref[...]
    xn = jnp.sum(x * x, axis=1)
    halves = []
    for h in range(2):
        best_v = jnp.full((NB,), -jnp.inf, jnp.float32)
        best_i = jnp.zeros((NB,), jnp.int32)
        for cc in range(K // KC // 2):
            c = h * (K // KC // 2) + cc
            e = et_ref[:, c * KC:(c + 1) * KC]
            en = jnp.sum(e * e, axis=0)
            conv = jnp.dot(x, e, preferred_element_type=jnp.float32)
            s = -((xn[:, None] - 2.0 * conv) + en[None, :])
            cm = jnp.max(s, axis=1)
            iota = lax.broadcasted_iota(jnp.int32, (NB, KC), 1)
            ci = jnp.min(jnp.where(s >= cm[:, None], iota, KC), axis=1) + c * KC
            upd = cm > best_v
            best_v = jnp.where(upd, cm, best_v)
            best_i = jnp.where(upd, ci, best_i)
        halves.append((best_v, best_i))
    (m0, i0), (m1, i1) = halves
    acc0 = m0.astype(jnp.bfloat16).astype(jnp.float32)
    upd = m1 > acc0
    best_i = jnp.where(upd, i1, i0)
    best_v = jnp.where(upd, m1, m0)
    ind_ref[...] = best_i.reshape(1, 1, NB)
    loss_ref[...] = jnp.sum(-best_v).reshape(1, 1, 1)


def _tc_argmin(x2d, et):
    return pl.pallas_call(
        _argmin_body,
        grid=(GRID,),
        in_specs=[
            pl.BlockSpec((NB, D), lambda i: (i, 0)),
            pl.BlockSpec((D, K), lambda i: (0, 0)),
        ],
        out_specs=[
            pl.BlockSpec((1, 1, NB), lambda i: (i, 0, 0)),
            pl.BlockSpec((1, 1, 1), lambda i: (i, 0, 0)),
        ],
        out_shape=[
            jax.ShapeDtypeStruct((GRID, 1, NB), jnp.int32),
            jax.ShapeDtypeStruct((GRID, 1, 1), jnp.float32),
        ],
    )(x2d, et)


def _gather_body(table_hbm, idx_hbm, out_hbm, idx_v, rows_v, sem):
    wid = lax.axis_index("s") * 2 + lax.axis_index("c")
    pltpu.sync_copy(idx_hbm.at[wid], idx_v)
    for j in range(_NCH):
        pltpu.async_copy(
            table_hbm.at[idx_v.at[j]],
            rows_v.at[pl.ds(j * _ICH, _ICH)],
            sem,
        ).wait()
    pltpu.sync_copy(rows_v, out_hbm.at[wid])


def _sc_gather(embed, idx3):
    mesh = plsc.VectorSubcoreMesh(core_axis_name="c", subcore_axis_name="s")
    f = pl.kernel(
        _gather_body,
        mesh=mesh,
        compiler_params=pltpu.CompilerParams(use_tc_tiling_on_sc=False),
        out_type=jax.ShapeDtypeStruct((_SC_NW, _BPW, D), jnp.float32),
        scratch_types=[
            pltpu.VMEM((_NCH, _ICH), jnp.int32),
            pltpu.VMEM((_BPW, D), jnp.float32),
            pltpu.SemaphoreType.DMA,
        ],
    )
    return f(embed, idx3)


def kernel(x, embed):
    shape = x.shape
    x2d = x.reshape(N, D)
    et = embed.T
    ind3, lossp = _tc_argmin(x2d, et)
    embed_ind = ind3.reshape(shape[:-1])
    commit_loss = jnp.sum(lossp) / (N * D)
    quantize = _sc_gather(embed, ind3.reshape(_SC_NW, _NCH, _ICH))
    return quantize.reshape(shape), embed_ind, commit_loss
